# baseline (speedup 1.0000x reference)
import functools

import jax
import jax.numpy as jnp
from jax import lax
from jax.experimental import pallas as pl
from jax.experimental.pallas import tpu as pltpu
from jax.experimental.pallas import tpu_sc as plsc

N = 10000
E = 320000
D_IN = 128
HID = 64
HEADS = 4
ACTIONS = 6

NP = 10240
EP = 331776
CH = 128
RT = NP // 16
R = 256
GRID = NP // R

_PREC = None


def _tc0_body(x_ref, w_ref, as_ref, ad_ref, hp_ref, aso_ref, ado_ref):
    h = jnp.dot(x_ref[...], w_ref[...], precision=_PREC,
                preferred_element_type=jnp.float32)
    hp_ref[0] = h[:, :128]
    hp_ref[1] = h[:, 128:]
    aso_ref[...] = jnp.dot(h, as_ref[...], precision=_PREC,
                           preferred_element_type=jnp.float32)
    ado_ref[...] = jnp.dot(h, ad_ref[...], precision=_PREC,
                           preferred_element_type=jnp.float32)


def _tc_layer0(x_pad, W0, As, Ad):
    return pl.pallas_call(
        _tc0_body,
        grid=(GRID,),
        in_specs=[
            pl.BlockSpec((R, D_IN), lambda i: (i, 0)),
            pl.BlockSpec((D_IN, 256), lambda i: (0, 0)),
            pl.BlockSpec((256, 16), lambda i: (0, 0)),
            pl.BlockSpec((256, 16), lambda i: (0, 0)),
        ],
        out_specs=[
            pl.BlockSpec((2, R, 128), lambda i: (0, i, 0)),
            pl.BlockSpec((R, 16), lambda i: (i, 0)),
            pl.BlockSpec((R, 16), lambda i: (i, 0)),
        ],
        out_shape=[
            jax.ShapeDtypeStruct((2, NP, 128), jnp.float32),
            jax.ShapeDtypeStruct((NP, 16), jnp.float32),
            jax.ShapeDtypeStruct((NP, 16), jnp.float32),
        ],
    )(x_pad, W0, As, Ad)


def _tc_mid_body(out_w, ag_ref, den_ref, b_ref, erep_ref, w_ref, as_ref,
                 ad_ref, h_ref, aso_ref, ado_ref):
    a = jnp.concatenate([ag_ref[0], ag_ref[1]], axis=1)
    den = jnp.dot(den_ref[0] + den_ref[1], erep_ref[...], precision=_PREC,
                  preferred_element_type=jnp.float32)
    v = a / (den + 1e-16) + b_ref[...]
    v = jnp.where(v > 0, v, jnp.exp(v) - 1.0)
    h = jnp.dot(v, w_ref[...], precision=_PREC,
                preferred_element_type=jnp.float32)
    if out_w == 256:
        h_ref[0] = h[:, :128]
        h_ref[1] = h[:, 128:]
    else:
        h_ref[...] = jnp.concatenate([h, jnp.zeros_like(h)], axis=1)
    aso_ref[...] = jnp.dot(h, as_ref[...], precision=_PREC,
                           preferred_element_type=jnp.float32)
    ado_ref[...] = jnp.dot(h, ad_ref[...], precision=_PREC,
                           preferred_element_type=jnp.float32)


def _tc_mid(agg, den, b_row, Erep, W, As, Ad, out_w):
    if out_w == 256:
        h_spec = pl.BlockSpec((2, R, 128), lambda i: (0, i, 0))
        h_shape = jax.ShapeDtypeStruct((2, NP, 128), jnp.float32)
    else:
        h_spec = pl.BlockSpec((R, 128), lambda i: (i, 0))
        h_shape = jax.ShapeDtypeStruct((NP, 128), jnp.float32)
    return pl.pallas_call(
        functools.partial(_tc_mid_body, out_w),
        grid=(GRID,),
        in_specs=[
            pl.BlockSpec((2, R, 128), lambda i: (0, i, 0)),
            pl.BlockSpec((2, R, 16), lambda i: (0, i, 0)),
            pl.BlockSpec((1, 256), lambda i: (0, 0)),
            pl.BlockSpec((16, 256), lambda i: (0, 0)),
            pl.BlockSpec((256, out_w), lambda i: (0, 0)),
            pl.BlockSpec((out_w, 16), lambda i: (0, 0)),
            pl.BlockSpec((out_w, 16), lambda i: (0, 0)),
        ],
        out_specs=[
            h_spec,
            pl.BlockSpec((R, 16), lambda i: (i, 0)),
            pl.BlockSpec((R, 16), lambda i: (i, 0)),
        ],
        out_shape=[
            h_shape,
            jax.ShapeDtypeStruct((NP, 16), jnp.float32),
            jax.ShapeDtypeStruct((NP, 16), jnp.float32),
        ],
    )(agg, den, b_row, Erep, W, As, Ad)


def _tc_mean_body(ag_ref, den_ref, b_ref, erep_ref, out_ref):
    i = pl.program_id(0)
    a = (ag_ref[0] + ag_ref[1])[:, :HID]
    d = den_ref[0] + den_ref[1]
    den = jnp.dot(d, erep_ref[...], precision=_PREC,
                  preferred_element_type=jnp.float32)
    h = a / (den + 1e-16) + b_ref[...]
    rows = i * R + lax.broadcasted_iota(jnp.int32, (R, 1), 0)
    h = jnp.where(rows < N, h, 0.0)
    s = jnp.sum(h, axis=0, keepdims=True)

    @pl.when(i == 0)
    def _():
        out_ref[...] = s

    @pl.when(i > 0)
    def _():
        out_ref[...] = out_ref[...] + s


def _tc_mean(aggp, denp, b2_row, Erep1):
    return pl.pallas_call(
        _tc_mean_body,
        grid=(GRID,),
        in_specs=[
            pl.BlockSpec((2, R, 128), lambda i: (0, i, 0)),
            pl.BlockSpec((2, R, 16), lambda i: (0, i, 0)),
            pl.BlockSpec((1, 64), lambda i: (0, 0)),
            pl.BlockSpec((16, 64), lambda i: (0, 0)),
        ],
        out_specs=pl.BlockSpec((1, 64), lambda i: (0, 0)),
        out_shape=jax.ShapeDtypeStruct((1, 64), jnp.float32),
    )(aggp, denp, b2_row, Erep1)


def _tc_heads_body(gs_ref, pw1_ref, pb1_ref, pw2_ref, pb2_ref,
                   vw1_ref, vb1_ref, vw2_ref, vb2_ref, lo_ref, va_ref):
    ge = gs_ref[...] * (1.0 / N)
    t = jnp.maximum(jnp.dot(ge, pw1_ref[...], precision=_PREC,
                            preferred_element_type=jnp.float32)
                    + pb1_ref[...], 0.0)
    lo_ref[...] = jnp.dot(t, pw2_ref[...], precision=_PREC,
                          preferred_element_type=jnp.float32) + pb2_ref[...]
    u = jnp.maximum(jnp.dot(ge, vw1_ref[...], precision=_PREC,
                            preferred_element_type=jnp.float32)
                    + vb1_ref[...], 0.0)
    va_ref[...] = jnp.dot(u, vw2_ref[...], precision=_PREC,
                          preferred_element_type=jnp.float32) + vb2_ref[...]


def _tc_heads(gesum, pW1, pb1, pW2, pb2, vW1, vb1, vW2, vb2):
    return pl.pallas_call(
        _tc_heads_body,
        out_shape=[
            jax.ShapeDtypeStruct((1, ACTIONS), jnp.float32),
            jax.ShapeDtypeStruct((1, 1), jnp.float32),
        ],
    )(gesum, pW1, pb1, pW2, pb2, vW1, vb1, vW2, vb2)


def _alpha_body(heads, src_h, dst_h, atabs_h, atabd_h, ex_h, den_h,
                denom_s, tabS, tabD, idx_s, idx_d, idx_dd, exbuf, dbuf):
    c = lax.axis_index("c")
    s = lax.axis_index("s")
    n_chunks = EP // 32 // CH
    iota16 = lax.iota(jnp.int32, 16)
    zeros16 = jnp.zeros((16,), jnp.float32)
    rowx = lax.shift_right_logical(iota16, 3)
    colx = (iota16 & 7) * 16

    pltpu.sync_copy(atabs_h, tabS)
    pltpu.sync_copy(atabd_h, tabD)

    def _z(j, _):
        for v in range(8):
            dbuf[j, pl.ds(v * 16, 16)] = zeros16
        return 0
    lax.fori_loop(0, CH, _z, 0)
    for j in range(CH // 8):
        for v in range(8):
            exbuf[j, pl.ds(v * 16, 16)] = zeros16
    rpt = NP // 8 // 16
    zbase = pl.multiple_of(s * rpt, 16)
    for k in range(rpt // (CH // 8)):
        pltpu.sync_copy(exbuf, denom_s.at[pl.ds(zbase + k * (CH // 8),
                                                CH // 8)])
    plsc.subcore_barrier()

    def _chunk(g, _):
        off = pl.multiple_of(c * (EP // 2) + (s * n_chunks + g) * CH, CH)
        pltpu.sync_copy(src_h.at[pl.ds(off, CH)], idx_s)
        pltpu.sync_copy(dst_h.at[pl.ds(off, CH)], idx_d)
        for q in range(CH // 16):
            sv = idx_s[pl.ds(q * 16, 16)]
            dv = idx_d[pl.ds(q * 16, 16)]
            idx_dd[pl.ds(q * 16, 16)] = lax.shift_right_logical(dv, 3)
            fs = sv * heads
            fd = dv * heads
            cold = (dv & 7) * 16
            jrow = q * 16 + iota16
            xrow = q * 2 + rowx
            for k in range(heads):
                a = plsc.load_gather(tabS, [fs + k])
                b = plsc.load_gather(tabD, [fd + k])
                e = a + b
                e = jnp.where(e >= 0.0, e, 0.2 * e)
                ex = jnp.exp(e)
                plsc.store_scatter(exbuf, [xrow, colx + k], ex)
                plsc.store_scatter(dbuf, [jrow, cold + k], ex)
        pltpu.sync_copy(exbuf,
                        ex_h.at[pl.ds(pl.multiple_of(off // 8, CH // 8),
                                      CH // 8)])
        pltpu.sync_copy(dbuf, denom_s.at[idx_dd], add=True)
        for q in range(CH // 16):
            dv = idx_d[pl.ds(q * 16, 16)]
            cold = (dv & 7) * 16
            jrow = q * 16 + iota16
            for k in range(heads):
                plsc.store_scatter(dbuf, [jrow, cold + k], zeros16)
        return 0

    lax.fori_loop(0, n_chunks, _chunk, 0)
    plsc.subcore_barrier()
    pltpu.sync_copy(denom_s.at[pl.ds(zbase, rpt)],
                    den_h.at[pl.ds(pl.multiple_of(c * (NP // 8) + s * rpt,
                                                  16), rpt)])


def _agg_body(feat, split_edges, src_h, dst_h, h_h, ex_h, agg_h,
              accum_s, idx_g, idx_d, exrow, hrows, sem_h, sem_w):
    c = lax.axis_index("c")
    s = lax.axis_index("s")
    nv = feat // 16
    n_chunks = (EP // 32 if split_edges else EP // 16) // CH
    zeros16 = jnp.zeros((16,), jnp.float32)

    def _z(j, _):
        for v in range(nv):
            hrows[0, j, pl.ds(v * 16, 16)] = zeros16
        return 0
    lax.fori_loop(0, CH, _z, 0)
    abase = pl.multiple_of(s * RT, CH)
    for k in range(RT // CH):
        pltpu.sync_copy(hrows.at[0], accum_s.at[pl.ds(abase + k * CH, CH)])
    plsc.subcore_barrier()

    def _chunk(g, _):
        b = g & 1
        if split_edges:
            off = c * (EP // 2) + (s * n_chunks + g) * CH
        else:
            off = (s * n_chunks + g) * CH
        off = pl.multiple_of(off, CH)
        @pl.when(g >= 2)
        def _():
            pltpu.make_async_copy(h_h.at[pl.ds(0, CH)], hrows.at[b],
                                  sem_w).wait()
        pltpu.sync_copy(src_h.at[pl.ds(off, CH)], idx_g)
        pltpu.sync_copy(dst_h.at[pl.ds(off, CH)], idx_d.at[b])
        if not split_edges:
            roff = c * NP
            for q in range(CH // 16):
                idx_g[pl.ds(q * 16, 16)] = idx_g[pl.ds(q * 16, 16)] + roff
        cph = pltpu.async_copy(h_h.at[idx_g], hrows.at[b], sem_h)
        pltpu.sync_copy(ex_h.at[pl.ds(pl.multiple_of(off // 8, CH // 8),
                                      CH // 8)], exrow)
        cph.wait()
        for j in range(CH):
            v = exrow[j // 8, pl.ds((j % 8) * 16, 16)]
            if split_edges:
                s0 = v[0]
                for q in range(nv):
                    hrows[b, j, pl.ds(q * 16, 16)] = (
                        hrows[b, j, pl.ds(q * 16, 16)] * s0)
            else:
                s0 = jnp.where(c == 0, v[0], v[2])
                s1 = jnp.where(c == 0, v[1], v[3])
                for q in range(4):
                    hrows[b, j, pl.ds(q * 16, 16)] = (
                        hrows[b, j, pl.ds(q * 16, 16)] * s0)
                for q in range(4, 8):
                    hrows[b, j, pl.ds(q * 16, 16)] = (
                        hrows[b, j, pl.ds(q * 16, 16)] * s1)
        pltpu.async_copy(hrows.at[b], accum_s.at[idx_d.at[b]], sem_w,
                         add=True)
        return 0

    lax.fori_loop(0, n_chunks, _chunk, 0)
    for b in range(2):
        pltpu.make_async_copy(h_h.at[pl.ds(0, CH)], hrows.at[b],
                              sem_w).wait()
    plsc.subcore_barrier()
    pltpu.sync_copy(accum_s.at[pl.ds(abase, RT)],
                    agg_h.at[pl.ds(pl.multiple_of(c * NP + s * RT, CH),
                                   RT)])


_SC_MESH = plsc.VectorSubcoreMesh(core_axis_name="c", subcore_axis_name="s")
_SC_PARAMS = pltpu.CompilerParams(needs_layout_passes=False)


def _make_alpha_call(heads):
    return pl.kernel(
        functools.partial(_alpha_body, heads),
        out_type=[
            jax.ShapeDtypeStruct((EP // 8, 128), jnp.float32),
            jax.ShapeDtypeStruct((2 * (NP // 8), 128), jnp.float32),
        ],
        mesh=_SC_MESH,
        compiler_params=_SC_PARAMS,
        scratch_types=[
            pltpu.VMEM_SHARED((NP // 8, 128), jnp.float32),
            pltpu.VMEM((NP * heads,), jnp.float32),
            pltpu.VMEM((NP * heads,), jnp.float32),
            pltpu.VMEM((CH,), jnp.int32),
            pltpu.VMEM((CH,), jnp.int32),
            pltpu.VMEM((CH,), jnp.int32),
            pltpu.VMEM((CH // 8, 128), jnp.float32),
            pltpu.VMEM((CH, 128), jnp.float32),
        ],
    )


def _make_agg_call(feat, split_edges):
    return pl.kernel(
        functools.partial(_agg_body, feat, split_edges),
        out_type=jax.ShapeDtypeStruct((2 * NP, feat), jnp.float32),
        mesh=_SC_MESH,
        compiler_params=_SC_PARAMS,
        scratch_types=[
            pltpu.VMEM_SHARED((NP, feat), jnp.float32),
            pltpu.VMEM((CH,), jnp.int32),
            pltpu.VMEM((2, CH), jnp.int32),
            pltpu.VMEM((CH // 8, 128), jnp.float32),
            pltpu.VMEM((2, CH, feat), jnp.float32),
            pltpu.SemaphoreType.DMA,
            pltpu.SemaphoreType.DMA,
        ],
    )


def _block_diag(a):
    h = a.shape[0]
    eye = jnp.eye(h, 16, dtype=jnp.float32)
    return (a[:, :, None] * eye[:, None, :]).reshape(h * HID, 16)


def _erep(heads, width):
    eye = jnp.eye(16, heads, dtype=jnp.float32)
    return jnp.repeat(eye, HID, axis=1)[:, : width]


def kernel(x, edge_index, W0, as0, ad0, b0, W1, as1, ad1, b1, W2, as2, ad2,
           b2, pW1, pb1, pW2, pb2, vW1, vb1, vW2, vb2):
    loop = jnp.arange(N, dtype=jnp.int32)
    pad = EP - E - N
    src = jnp.concatenate([edge_index[0], loop,
                           jnp.zeros((pad,), jnp.int32)])
    dst = jnp.concatenate([edge_index[1], loop,
                           jnp.full((pad,), N, jnp.int32)])
    x_pad = jnp.zeros((NP, D_IN), jnp.float32).at[:N].set(x)

    As0, Ad0 = _block_diag(as0), _block_diag(ad0)
    As1, Ad1 = _block_diag(as1), _block_diag(ad1)
    As2, Ad2 = _block_diag(as2), _block_diag(ad2)
    Erep4 = _erep(HEADS, 256)
    Erep1 = _erep(1, 64)
    b0_row = b0.reshape(1, 256)
    b1_row = b1.reshape(1, 256)
    b2_row = b2.reshape(1, 64)

    alpha4 = _make_alpha_call(HEADS)
    alpha1 = _make_alpha_call(1)
    agg_w = _make_agg_call(128, False)
    agg_2 = _make_agg_call(128, True)

    hp0, ats0, atd0 = _tc_layer0(x_pad, W0, As0, Ad0)
    ex0, den0p = alpha4(src, dst, ats0[:, :HEADS].reshape(-1),
                        atd0[:, :HEADS].reshape(-1))
    agg0 = agg_w(src, dst, hp0.reshape(2 * NP, 128), ex0)

    hp1, ats1, atd1 = _tc_mid(agg0.reshape(2, NP, 128),
                              den0p.reshape(2, NP, 16),
                              b0_row, Erep4, W1, As1, Ad1, 256)
    ex1, den1p = alpha4(src, dst, ats1[:, :HEADS].reshape(-1),
                        atd1[:, :HEADS].reshape(-1))
    agg1 = agg_w(src, dst, hp1.reshape(2 * NP, 128), ex1)

    h2, ats2, atd2 = _tc_mid(agg1.reshape(2, NP, 128),
                             den1p.reshape(2, NP, 16),
                             b1_row, Erep4, W2, As2, Ad2, 64)
    ex2, den2p = alpha1(src, dst, ats2[:, :1].reshape(-1),
                        atd2[:, :1].reshape(-1))
    agg2 = agg_2(src, dst, h2, ex2)

    gesum = _tc_mean(agg2.reshape(2, NP, 128), den2p.reshape(2, NP, 16),
                     b2_row, Erep1)
    logits, value = _tc_heads(gesum, pW1, pb1.reshape(1, HID),
                              pW2, pb2.reshape(1, ACTIONS),
                              vW1, vb1.reshape(1, HID),
                              vW2, vb2.reshape(1, 1))
    return (logits, value)


# speedup vs baseline: 1.2901x; 1.2901x over previous
import functools

import jax
import jax.numpy as jnp
from jax import lax
from jax.experimental import pallas as pl
from jax.experimental.pallas import tpu as pltpu
from jax.experimental.pallas import tpu_sc as plsc

N = 10000
E = 320000
D_IN = 128
HID = 64
HEADS = 4
ACTIONS = 6

NP = 10240
EP = 331776
CH = 128
RT = NP // 16
R = 256
GRID = NP // R

_PREC = None


def _tc0_body(x_ref, w_ref, as_ref, ad_ref, hp_ref, aso_ref, ado_ref):
    h = jnp.dot(x_ref[...], w_ref[...], precision=_PREC,
                preferred_element_type=jnp.float32)
    hp_ref[0] = h[:, :128]
    hp_ref[1] = h[:, 128:]
    aso_ref[...] = jnp.dot(h, as_ref[...], precision=_PREC,
                           preferred_element_type=jnp.float32)
    ado_ref[...] = jnp.dot(h, ad_ref[...], precision=_PREC,
                           preferred_element_type=jnp.float32)


def _tc_layer0(x_pad, W0, As, Ad):
    return pl.pallas_call(
        _tc0_body,
        grid=(GRID,),
        in_specs=[
            pl.BlockSpec((R, D_IN), lambda i: (i, 0)),
            pl.BlockSpec((D_IN, 256), lambda i: (0, 0)),
            pl.BlockSpec((256, 16), lambda i: (0, 0)),
            pl.BlockSpec((256, 16), lambda i: (0, 0)),
        ],
        out_specs=[
            pl.BlockSpec((2, R, 128), lambda i: (0, i, 0)),
            pl.BlockSpec((R, 16), lambda i: (i, 0)),
            pl.BlockSpec((R, 16), lambda i: (i, 0)),
        ],
        out_shape=[
            jax.ShapeDtypeStruct((2, NP, 128), jnp.float32),
            jax.ShapeDtypeStruct((NP, 16), jnp.float32),
            jax.ShapeDtypeStruct((NP, 16), jnp.float32),
        ],
    )(x_pad, W0, As, Ad)


def _tc_mid_body(out_w, ag_ref, den_ref, b_ref, erep_ref, w_ref, as_ref,
                 ad_ref, h_ref, aso_ref, ado_ref):
    a = jnp.concatenate([ag_ref[0], ag_ref[1]], axis=1)
    den = jnp.dot(den_ref[0] + den_ref[1], erep_ref[...], precision=_PREC,
                  preferred_element_type=jnp.float32)
    v = a / (den + 1e-16) + b_ref[...]
    v = jnp.where(v > 0, v, jnp.exp(v) - 1.0)
    h = jnp.dot(v, w_ref[...], precision=_PREC,
                preferred_element_type=jnp.float32)
    if out_w == 256:
        h_ref[0] = h[:, :128]
        h_ref[1] = h[:, 128:]
    else:
        h_ref[...] = jnp.concatenate([h, jnp.zeros_like(h)], axis=1)
    aso_ref[...] = jnp.dot(h, as_ref[...], precision=_PREC,
                           preferred_element_type=jnp.float32)
    ado_ref[...] = jnp.dot(h, ad_ref[...], precision=_PREC,
                           preferred_element_type=jnp.float32)


def _tc_mid(agg, den, b_row, Erep, W, As, Ad, out_w):
    if out_w == 256:
        h_spec = pl.BlockSpec((2, R, 128), lambda i: (0, i, 0))
        h_shape = jax.ShapeDtypeStruct((2, NP, 128), jnp.float32)
    else:
        h_spec = pl.BlockSpec((R, 128), lambda i: (i, 0))
        h_shape = jax.ShapeDtypeStruct((NP, 128), jnp.float32)
    return pl.pallas_call(
        functools.partial(_tc_mid_body, out_w),
        grid=(GRID,),
        in_specs=[
            pl.BlockSpec((2, R, 128), lambda i: (0, i, 0)),
            pl.BlockSpec((2, R, 16), lambda i: (0, i, 0)),
            pl.BlockSpec((1, 256), lambda i: (0, 0)),
            pl.BlockSpec((16, 256), lambda i: (0, 0)),
            pl.BlockSpec((256, out_w), lambda i: (0, 0)),
            pl.BlockSpec((out_w, 16), lambda i: (0, 0)),
            pl.BlockSpec((out_w, 16), lambda i: (0, 0)),
        ],
        out_specs=[
            h_spec,
            pl.BlockSpec((R, 16), lambda i: (i, 0)),
            pl.BlockSpec((R, 16), lambda i: (i, 0)),
        ],
        out_shape=[
            h_shape,
            jax.ShapeDtypeStruct((NP, 16), jnp.float32),
            jax.ShapeDtypeStruct((NP, 16), jnp.float32),
        ],
    )(agg, den, b_row, Erep, W, As, Ad)


def _tc_mean_body(ag_ref, den_ref, b_ref, erep_ref, out_ref):
    i = pl.program_id(0)
    a = (ag_ref[0] + ag_ref[1])[:, :HID]
    d = den_ref[0] + den_ref[1]
    den = jnp.dot(d, erep_ref[...], precision=_PREC,
                  preferred_element_type=jnp.float32)
    h = a / (den + 1e-16) + b_ref[...]
    rows = i * R + lax.broadcasted_iota(jnp.int32, (R, 1), 0)
    h = jnp.where(rows < N, h, 0.0)
    s = jnp.sum(h, axis=0, keepdims=True)

    @pl.when(i == 0)
    def _():
        out_ref[...] = s

    @pl.when(i > 0)
    def _():
        out_ref[...] = out_ref[...] + s


def _tc_mean(aggp, denp, b2_row, Erep1):
    return pl.pallas_call(
        _tc_mean_body,
        grid=(GRID,),
        in_specs=[
            pl.BlockSpec((2, R, 128), lambda i: (0, i, 0)),
            pl.BlockSpec((2, R, 16), lambda i: (0, i, 0)),
            pl.BlockSpec((1, 64), lambda i: (0, 0)),
            pl.BlockSpec((16, 64), lambda i: (0, 0)),
        ],
        out_specs=pl.BlockSpec((1, 64), lambda i: (0, 0)),
        out_shape=jax.ShapeDtypeStruct((1, 64), jnp.float32),
    )(aggp, denp, b2_row, Erep1)


def _tc_heads_body(gs_ref, pw1_ref, pb1_ref, pw2_ref, pb2_ref,
                   vw1_ref, vb1_ref, vw2_ref, vb2_ref, lo_ref, va_ref):
    ge = gs_ref[...] * (1.0 / N)
    t = jnp.maximum(jnp.dot(ge, pw1_ref[...], precision=_PREC,
                            preferred_element_type=jnp.float32)
                    + pb1_ref[...], 0.0)
    lo_ref[...] = jnp.dot(t, pw2_ref[...], precision=_PREC,
                          preferred_element_type=jnp.float32) + pb2_ref[...]
    u = jnp.maximum(jnp.dot(ge, vw1_ref[...], precision=_PREC,
                            preferred_element_type=jnp.float32)
                    + vb1_ref[...], 0.0)
    va_ref[...] = jnp.dot(u, vw2_ref[...], precision=_PREC,
                          preferred_element_type=jnp.float32) + vb2_ref[...]


def _tc_heads(gesum, pW1, pb1, pW2, pb2, vW1, vb1, vW2, vb2):
    return pl.pallas_call(
        _tc_heads_body,
        out_shape=[
            jax.ShapeDtypeStruct((1, ACTIONS), jnp.float32),
            jax.ShapeDtypeStruct((1, 1), jnp.float32),
        ],
    )(gesum, pW1, pb1, pW2, pb2, vW1, vb1, vW2, vb2)


def _alpha_body(heads, src_h, dst_h, atabs_h, atabd_h, ex_h, den_h,
                denom_s, tabS, tabD, idx_s, idx_d, idx_dd, exbuf, dbuf):
    c = lax.axis_index("c")
    s = lax.axis_index("s")
    n_chunks = EP // 32 // CH
    iota16 = lax.iota(jnp.int32, 16)
    zeros16 = jnp.zeros((16,), jnp.float32)
    rowx = lax.shift_right_logical(iota16, 3)
    colx = (iota16 & 7) * 16

    pltpu.sync_copy(atabs_h, tabS)
    pltpu.sync_copy(atabd_h, tabD)

    def _z(j, _):
        for v in range(8):
            dbuf[j, pl.ds(v * 16, 16)] = zeros16
        return 0
    lax.fori_loop(0, CH, _z, 0)
    for j in range(CH // 8):
        for v in range(8):
            exbuf[j, pl.ds(v * 16, 16)] = zeros16
    rpt = NP // 8 // 16
    zbase = pl.multiple_of(s * rpt, 16)
    for k in range(rpt // (CH // 8)):
        pltpu.sync_copy(exbuf, denom_s.at[pl.ds(zbase + k * (CH // 8),
                                                CH // 8)])
    plsc.subcore_barrier()

    def _chunk(g, _):
        off = pl.multiple_of(c * (EP // 2) + (s * n_chunks + g) * CH, CH)
        pltpu.sync_copy(src_h.at[pl.ds(off, CH)], idx_s)
        pltpu.sync_copy(dst_h.at[pl.ds(off, CH)], idx_d)
        for q in range(CH // 16):
            sv = idx_s[pl.ds(q * 16, 16)]
            dv = idx_d[pl.ds(q * 16, 16)]
            idx_dd[pl.ds(q * 16, 16)] = lax.shift_right_logical(dv, 3)
            fs = sv * heads
            fd = dv * heads
            cold = (dv & 7) * 16
            jrow = q * 16 + iota16
            xrow = q * 2 + rowx
            for k in range(heads):
                a = plsc.load_gather(tabS, [fs + k])
                b = plsc.load_gather(tabD, [fd + k])
                e = a + b
                e = jnp.where(e >= 0.0, e, 0.2 * e)
                ex = jnp.exp(e)
                plsc.store_scatter(exbuf, [xrow, colx + k], ex)
                plsc.store_scatter(dbuf, [jrow, cold + k], ex)
        pltpu.sync_copy(exbuf,
                        ex_h.at[pl.ds(pl.multiple_of(off // 8, CH // 8),
                                      CH // 8)])
        pltpu.sync_copy(dbuf, denom_s.at[idx_dd], add=True)
        for q in range(CH // 16):
            dv = idx_d[pl.ds(q * 16, 16)]
            cold = (dv & 7) * 16
            jrow = q * 16 + iota16
            for k in range(heads):
                plsc.store_scatter(dbuf, [jrow, cold + k], zeros16)
        return 0

    lax.fori_loop(0, n_chunks, _chunk, 0)
    plsc.subcore_barrier()
    pltpu.sync_copy(denom_s.at[pl.ds(zbase, rpt)],
                    den_h.at[pl.ds(pl.multiple_of(c * (NP // 8) + s * rpt,
                                                  16), rpt)])


def _agg_body(feat, split_edges, src_h, dst_h, h_h, ex_h, agg_h,
              accum_s, idx_g, idx_d, exrow, hrows, sem_h):
    c = lax.axis_index("c")
    s = lax.axis_index("s")
    nv = feat // 16
    n_chunks = (EP // 32 if split_edges else EP // 16) // CH
    zeros16 = jnp.zeros((16,), jnp.float32)

    def _z(j, _):
        for v in range(nv):
            hrows[j, pl.ds(v * 16, 16)] = zeros16
        return 0
    lax.fori_loop(0, CH, _z, 0)
    abase = pl.multiple_of(s * RT, CH)
    for k in range(RT // CH):
        pltpu.sync_copy(hrows, accum_s.at[pl.ds(abase + k * CH, CH)])
    plsc.subcore_barrier()

    def _chunk(g, _):
        if split_edges:
            off = c * (EP // 2) + (s * n_chunks + g) * CH
        else:
            off = (s * n_chunks + g) * CH
        off = pl.multiple_of(off, CH)
        pltpu.sync_copy(src_h.at[pl.ds(off, CH)], idx_g)
        pltpu.sync_copy(dst_h.at[pl.ds(off, CH)], idx_d)
        if not split_edges:
            roff = c * NP
            for q in range(CH // 16):
                idx_g[pl.ds(q * 16, 16)] = idx_g[pl.ds(q * 16, 16)] + roff
        cph = pltpu.async_copy(h_h.at[idx_g], hrows, sem_h)
        pltpu.sync_copy(ex_h.at[pl.ds(pl.multiple_of(off // 8, CH // 8),
                                      CH // 8)], exrow)
        cph.wait()
        for j in range(CH):
            v = exrow[j // 8, pl.ds((j % 8) * 16, 16)]
            if split_edges:
                s0 = v[0]
                for q in range(nv):
                    hrows[j, pl.ds(q * 16, 16)] = (
                        hrows[j, pl.ds(q * 16, 16)] * s0)
            else:
                s0 = jnp.where(c == 0, v[0], v[2])
                s1 = jnp.where(c == 0, v[1], v[3])
                for q in range(4):
                    hrows[j, pl.ds(q * 16, 16)] = (
                        hrows[j, pl.ds(q * 16, 16)] * s0)
                for q in range(4, 8):
                    hrows[j, pl.ds(q * 16, 16)] = (
                        hrows[j, pl.ds(q * 16, 16)] * s1)
        pltpu.sync_copy(hrows, accum_s.at[idx_d], add=True)
        return 0

    lax.fori_loop(0, n_chunks, _chunk, 0)
    plsc.subcore_barrier()
    pltpu.sync_copy(accum_s.at[pl.ds(abase, RT)],
                    agg_h.at[pl.ds(pl.multiple_of(c * NP + s * RT, CH),
                                   RT)])


_SC_MESH = plsc.VectorSubcoreMesh(core_axis_name="c", subcore_axis_name="s")
_SC_PARAMS = pltpu.CompilerParams(needs_layout_passes=False)


def _make_alpha_call(heads):
    return pl.kernel(
        functools.partial(_alpha_body, heads),
        out_type=[
            jax.ShapeDtypeStruct((EP // 8, 128), jnp.float32),
            jax.ShapeDtypeStruct((2 * (NP // 8), 128), jnp.float32),
        ],
        mesh=_SC_MESH,
        compiler_params=_SC_PARAMS,
        scratch_types=[
            pltpu.VMEM_SHARED((NP // 8, 128), jnp.float32),
            pltpu.VMEM((NP * heads,), jnp.float32),
            pltpu.VMEM((NP * heads,), jnp.float32),
            pltpu.VMEM((CH,), jnp.int32),
            pltpu.VMEM((CH,), jnp.int32),
            pltpu.VMEM((CH,), jnp.int32),
            pltpu.VMEM((CH // 8, 128), jnp.float32),
            pltpu.VMEM((CH, 128), jnp.float32),
        ],
    )


def _make_agg_call(feat, split_edges):
    return pl.kernel(
        functools.partial(_agg_body, feat, split_edges),
        out_type=jax.ShapeDtypeStruct((2 * NP, feat), jnp.float32),
        mesh=_SC_MESH,
        compiler_params=_SC_PARAMS,
        scratch_types=[
            pltpu.VMEM_SHARED((NP, feat), jnp.float32),
            pltpu.VMEM((CH,), jnp.int32),
            pltpu.VMEM((CH,), jnp.int32),
            pltpu.VMEM((CH // 8, 128), jnp.float32),
            pltpu.VMEM((CH, feat), jnp.float32),
            pltpu.SemaphoreType.DMA,
        ],
    )


def _block_diag(a):
    h = a.shape[0]
    eye = jnp.eye(h, 16, dtype=jnp.float32)
    return (a[:, :, None] * eye[:, None, :]).reshape(h * HID, 16)


def _erep(heads, width):
    eye = jnp.eye(16, heads, dtype=jnp.float32)
    return jnp.repeat(eye, HID, axis=1)[:, : width]


def kernel(x, edge_index, W0, as0, ad0, b0, W1, as1, ad1, b1, W2, as2, ad2,
           b2, pW1, pb1, pW2, pb2, vW1, vb1, vW2, vb2):
    loop = jnp.arange(N, dtype=jnp.int32)
    pad = EP - E - N
    src = jnp.concatenate([edge_index[0], loop,
                           jnp.zeros((pad,), jnp.int32)])
    dst = jnp.concatenate([edge_index[1], loop,
                           jnp.full((pad,), N, jnp.int32)])
    x_pad = jnp.zeros((NP, D_IN), jnp.float32).at[:N].set(x)

    As0, Ad0 = _block_diag(as0), _block_diag(ad0)
    As1, Ad1 = _block_diag(as1), _block_diag(ad1)
    As2, Ad2 = _block_diag(as2), _block_diag(ad2)
    Erep4 = _erep(HEADS, 256)
    Erep1 = _erep(1, 64)
    b0_row = b0.reshape(1, 256)
    b1_row = b1.reshape(1, 256)
    b2_row = b2.reshape(1, 64)

    alpha4 = _make_alpha_call(HEADS)
    alpha1 = _make_alpha_call(1)
    agg_w = _make_agg_call(128, False)
    agg_2 = _make_agg_call(128, True)

    hp0, ats0, atd0 = _tc_layer0(x_pad, W0, As0, Ad0)
    ex0, den0p = alpha4(src, dst, ats0[:, :HEADS].reshape(-1),
                        atd0[:, :HEADS].reshape(-1))
    agg0 = agg_w(src, dst, hp0.reshape(2 * NP, 128), ex0)

    hp1, ats1, atd1 = _tc_mid(agg0.reshape(2, NP, 128),
                              den0p.reshape(2, NP, 16),
                              b0_row, Erep4, W1, As1, Ad1, 256)
    ex1, den1p = alpha4(src, dst, ats1[:, :HEADS].reshape(-1),
                        atd1[:, :HEADS].reshape(-1))
    agg1 = agg_w(src, dst, hp1.reshape(2 * NP, 128), ex1)

    h2, ats2, atd2 = _tc_mid(agg1.reshape(2, NP, 128),
                             den1p.reshape(2, NP, 16),
                             b1_row, Erep4, W2, As2, Ad2, 64)
    ex2, den2p = alpha1(src, dst, ats2[:, :1].reshape(-1),
                        atd2[:, :1].reshape(-1))
    agg2 = agg_2(src, dst, h2, ex2)

    gesum = _tc_mean(agg2.reshape(2, NP, 128), den2p.reshape(2, NP, 16),
                     b2_row, Erep1)
    logits, value = _tc_heads(gesum, pW1, pb1.reshape(1, HID),
                              pW2, pb2.reshape(1, ACTIONS),
                              vW1, vb1.reshape(1, HID),
                              vW2, vb2.reshape(1, 1))
    return (logits, value)
